# baseline (speedup 1.0000x reference)
import jax
import jax.numpy as jnp
from jax import lax
from jax.experimental import pallas as pl
from jax.experimental.pallas import tpu as pltpu
from jax.experimental.pallas import tpu_sc as plsc

_NT = 32
_T = 32
_B = 8
_V = 10000


def _emit_sc(emitf_hbm, x_hbm, g_hbm, stats_hbm, row_v, idx_v, gout_v, stat_v,
             rsem, gsem):
    a = lax.axis_index("s") * 2 + lax.axis_index("c")
    rowcp = pltpu.async_copy(emitf_hbm.at[pl.ds(a * _V, _V)], row_v, rsem)
    pltpu.sync_copy(x_hbm, idx_v)

    for j in range(_T * _B // 16):
        idx_v[pl.ds(j * 16, 16)] = idx_v[pl.ds(j * 16, 16)] + a * _V
    copies = [
        pltpu.async_copy(emitf_hbm.at[idx_v.at[pl.ds(c * 128, 128)]],
                         gout_v.at[pl.ds(c * 128, 128)], gsem)
        for c in range(_T * _B // 128)
    ]
    rowcp.wait()

    unroll = 25
    n_it = _V // 16 // unroll

    def mbody(i, acc):
        for u in range(unroll):
            acc = jnp.maximum(acc, row_v[pl.ds(i * (16 * unroll) + u * 16, 16)])
        return acc
    macc = lax.fori_loop(0, n_it, mbody,
                         jnp.full((16,), -jnp.inf, jnp.float32))

    def sbody(i, acc):
        for u in range(unroll):
            acc = acc + jnp.exp(
                row_v[pl.ds(i * (16 * unroll) + u * 16, 16)] - macc)
        return acc
    sacc = lax.fori_loop(0, n_it, sbody, jnp.zeros((16,), jnp.float32))

    stat_v[pl.ds(0, 16)] = macc
    stat_v[pl.ds(16, 16)] = sacc
    pltpu.sync_copy(stat_v, stats_hbm.at[a])

    for c in copies:
        c.wait()
    pltpu.sync_copy(gout_v, g_hbm.at[a])


_emit_call_cache = []


def _emit_call(emitf, xt):
    if not _emit_call_cache:
        _emit_call_cache.append(pl.kernel(
            _emit_sc,
            out_type=(
                jax.ShapeDtypeStruct((_NT, _T * _B), jnp.float32),
                jax.ShapeDtypeStruct((_NT, 32), jnp.float32),
            ),
            mesh=plsc.VectorSubcoreMesh(core_axis_name="c",
                                        subcore_axis_name="s"),
            scratch_types=[
                pltpu.VMEM((_V,), jnp.float32),
                pltpu.VMEM((_T * _B,), jnp.int32),
                pltpu.VMEM((_T * _B,), jnp.float32),
                pltpu.VMEM((32,), jnp.float32),
                pltpu.SemaphoreType.DMA,
                pltpu.SemaphoreType.DMA,
            ],
        ))
    return _emit_call_cache[0](emitf, xt)


def _inside_kernel(g_ref, stats_ref, root_ref, rule_ref, out_ref,
                   es_ref, er_ref):
    B, T, NT = _B, _T, _NT
    NN = NT * NT

    stats = stats_ref[...]
    tmb = stats[:, 0:16] + jnp.log(stats[:, 16:32])
    tmm = jnp.max(tmb, axis=1, keepdims=True)
    logz = tmm + jnp.log(jnp.sum(jnp.exp(tmb - tmm), axis=1, keepdims=True))
    gn = g_ref[...] - logz
    rown = jax.lax.broadcasted_iota(jnp.int32, (NT, NT), 0)
    coln = jax.lax.broadcasted_iota(jnp.int32, (NT, NT), 1)
    eye = (rown == coln).astype(jnp.float32)
    beta1 = jax.lax.dot_general(gn, eye, (((0,), (0,)), ((), ())),
                                preferred_element_type=jnp.float32,
                                precision=jax.lax.Precision.HIGHEST)
    m1 = jnp.max(jnp.max(beta1, axis=1, keepdims=True), axis=0,
                 keepdims=True)
    e1 = jnp.exp(beta1 - m1)

    rl = rule_ref[...]
    rm = jnp.max(rl, axis=1, keepdims=True)
    re = jnp.exp(rl - rm)
    rprob = re / jnp.sum(re, axis=1, keepdims=True)

    jj = jax.lax.broadcasted_iota(jnp.int32, (NT, NN), 1)
    row = jax.lax.broadcasted_iota(jnp.int32, (NT, NN), 0)
    tilemat = (jj % NT == row).astype(jnp.float32)
    repmat = (jj // NT == row).astype(jnp.float32)

    es_ref[1] = e1
    er_ref[T] = e1
    mv = [None, m1]

    root = root_ref[...]
    rt = jnp.exp(root - jnp.max(root, axis=1, keepdims=True))
    rsm = rt / jnp.sum(rt, axis=1, keepdims=True)

    for w in range(2, T + 1):
        n8 = (T - w + 1) * B
        lo = T + 1 - w
        k = w - 1
        ls = es_ref[1:w, 0:n8, :]
        rs = er_ref[lo + 1:T + 1, (w - 1) * B:T * B, :]
        sk = jnp.concatenate([mv[i] + mv[w - i] for i in range(1, w)],
                             axis=0)
        s = jnp.max(sk, axis=0, keepdims=True)
        scale = jnp.exp(sk - s)[:, :, None]
        lw = (ls * scale).reshape(k * n8, NT)
        lt = jnp.dot(lw, tilemat, preferred_element_type=jnp.float32)
        rr = jnp.dot(rs.reshape(k * n8, NT), repmat,
                     preferred_element_type=jnp.float32)
        vk = jax.lax.dot_general(lt * rr, rprob, (((1,), (1,)), ((), ())),
                                 preferred_element_type=jnp.float32)
        v = jnp.sum(vk.reshape(k, n8, NT), axis=0)
        if w < T:
            vmax = jnp.max(jnp.max(v, axis=1, keepdims=True), axis=0,
                           keepdims=True)
            ew = v * (1.0 / vmax)
            mv.append(s + jnp.log(vmax))
            es_ref[w, 0:n8, :] = ew
            er_ref[lo, (w - 1) * B:T * B, :] = ew
        else:
            acc = jnp.sum(v * rsm, axis=1, keepdims=True)
            out_ref[...] = s + jnp.log(acc)


def kernel(x, root_logits, rule_logits, emit_logits):
    xt = x.astype(jnp.int32).T.reshape(_T * _B)
    g, stats = _emit_call(emit_logits.reshape(_NT * _V), xt)
    root2 = root_logits.reshape(1, _NT)
    rule2 = rule_logits.transpose(0, 2, 1).reshape(_NT, _NT * _NT)
    ll = pl.pallas_call(
        _inside_kernel,
        out_shape=jax.ShapeDtypeStruct((_B, 1), jnp.float32),
        scratch_shapes=[
            pltpu.VMEM((_T + 1, _T * _B, _NT), jnp.float32),
            pltpu.VMEM((_T + 1, _T * _B, _NT), jnp.float32),
        ],
    )(g, stats, root2, rule2)
    return ll.reshape(_B)


# speedup vs baseline: 2.2925x; 2.2925x over previous
import jax
import jax.numpy as jnp
from jax import lax
from jax.experimental import pallas as pl
from jax.experimental.pallas import tpu as pltpu
from jax.experimental.pallas import tpu_sc as plsc

_NT = 32
_T = 32
_B = 8
_V = 10000


def _emit_sc(emitf_hbm, x_hbm, g_hbm, stats_hbm, row_v, idx_v, gout_v, stat_v,
             rsem, gsem):
    a = lax.axis_index("s") * 2 + lax.axis_index("c")
    rowcp = pltpu.async_copy(emitf_hbm.at[pl.ds(a * _V, _V)], row_v, rsem)
    pltpu.sync_copy(x_hbm, idx_v)

    for j in range(_T * _B // 16):
        idx_v[pl.ds(j * 16, 16)] = idx_v[pl.ds(j * 16, 16)] + a * _V
    copies = [
        pltpu.async_copy(emitf_hbm.at[idx_v.at[pl.ds(c * 128, 128)]],
                         gout_v.at[pl.ds(c * 128, 128)], gsem)
        for c in range(_T * _B // 128)
    ]
    rowcp.wait()

    unroll = 25
    n_it = _V // 16 // unroll

    def mbody(i, acc):
        for u in range(unroll):
            acc = jnp.maximum(acc, row_v[pl.ds(i * (16 * unroll) + u * 16, 16)])
        return acc
    macc = lax.fori_loop(0, n_it, mbody,
                         jnp.full((16,), -jnp.inf, jnp.float32))

    def sbody(i, acc):
        for u in range(unroll):
            acc = acc + jnp.exp(
                row_v[pl.ds(i * (16 * unroll) + u * 16, 16)] - macc)
        return acc
    sacc = lax.fori_loop(0, n_it, sbody, jnp.zeros((16,), jnp.float32))

    stat_v[pl.ds(0, 16)] = macc
    stat_v[pl.ds(16, 16)] = sacc
    pltpu.sync_copy(stat_v, stats_hbm.at[a])

    for c in copies:
        c.wait()
    pltpu.sync_copy(gout_v, g_hbm.at[a])


_emit_call_cache = []


def _emit_call(emitf, xt):
    if not _emit_call_cache:
        _emit_call_cache.append(pl.kernel(
            _emit_sc,
            out_type=(
                jax.ShapeDtypeStruct((_NT, _T * _B), jnp.float32),
                jax.ShapeDtypeStruct((_NT, 32), jnp.float32),
            ),
            mesh=plsc.VectorSubcoreMesh(core_axis_name="c",
                                        subcore_axis_name="s"),
            scratch_types=[
                pltpu.VMEM((_V,), jnp.float32),
                pltpu.VMEM((_T * _B,), jnp.int32),
                pltpu.VMEM((_T * _B,), jnp.float32),
                pltpu.VMEM((32,), jnp.float32),
                pltpu.SemaphoreType.DMA,
                pltpu.SemaphoreType.DMA,
            ],
        ))
    return _emit_call_cache[0](emitf, xt)


def _inside_kernel(g_ref, stats_ref, root_ref, rule_ref, out_ref, ch_ref):
    B, T, NT = _B, _T, _NT
    NN = NT * NT

    stats = stats_ref[...]
    tmb = stats[:, 0:16] + jnp.log(stats[:, 16:32])
    tmm = jnp.max(tmb, axis=1, keepdims=True)
    logz = tmm + jnp.log(jnp.sum(jnp.exp(tmb - tmm), axis=1, keepdims=True))
    gn = g_ref[...] - logz
    rown = jax.lax.broadcasted_iota(jnp.int32, (NT, NT), 0)
    coln = jax.lax.broadcasted_iota(jnp.int32, (NT, NT), 1)
    eye = (rown == coln).astype(jnp.float32)
    beta1 = jax.lax.dot_general(gn, eye, (((0,), (0,)), ((), ())),
                                preferred_element_type=jnp.float32,
                                precision=jax.lax.Precision.HIGHEST)
    m1 = jnp.max(jnp.max(beta1, axis=1, keepdims=True), axis=0,
                 keepdims=True)
    e1 = jnp.exp(beta1 - m1)

    rl = rule_ref[...]
    rm = jnp.max(rl, axis=1, keepdims=True)
    re = jnp.exp(rl - rm)
    rprob = re / jnp.sum(re, axis=1, keepdims=True)

    jj = jax.lax.broadcasted_iota(jnp.int32, (NT, NN), 1)
    row = jax.lax.broadcasted_iota(jnp.int32, (NT, NN), 0)
    tilemat = (jj % NT == row).astype(jnp.float32)
    repmat = (jj // NT == row).astype(jnp.float32)

    ch_ref[1, 0:T * B, :] = jnp.dot(e1, tilemat,
                                    preferred_element_type=jnp.float32)
    ch_ref[T, B:(T + 1) * B, :] = jnp.dot(e1, repmat,
                                          preferred_element_type=jnp.float32)
    mv = [None, m1]

    root = root_ref[...]
    rt = jnp.exp(root - jnp.max(root, axis=1, keepdims=True))
    rsm = rt / jnp.sum(rt, axis=1, keepdims=True)

    for w in range(2, T + 1):
        n8 = (T - w + 1) * B
        lo = T + 1 - w
        k = w - 1
        ls = ch_ref[1:w, 0:n8, :]
        rs = ch_ref[lo + 1:T + 1, w * B:(T + 1) * B, :]
        sk = jnp.concatenate([mv[i] + mv[w - i] for i in range(1, w)],
                             axis=0)
        s = jnp.max(sk, axis=0, keepdims=True)
        scale = jnp.exp(sk - s)[:, :, None]
        c = jnp.sum(ls * rs * scale, axis=0)
        v = jax.lax.dot_general(c, rprob, (((1,), (1,)), ((), ())),
                                preferred_element_type=jnp.float32)
        if w < T:
            vmax = jnp.max(jnp.max(v, axis=1, keepdims=True), axis=0,
                           keepdims=True)
            ew = v * (1.0 / vmax)
            mv.append(s + jnp.log(vmax))
            ch_ref[w, 0:n8, :] = jnp.dot(ew, tilemat,
                                         preferred_element_type=jnp.float32)
            ch_ref[lo, w * B:(T + 1) * B, :] = jnp.dot(
                ew, repmat, preferred_element_type=jnp.float32)
        else:
            acc = jnp.sum(v * rsm, axis=1, keepdims=True)
            out_ref[...] = s + jnp.log(acc)


def kernel(x, root_logits, rule_logits, emit_logits):
    xt = x.astype(jnp.int32).T.reshape(_T * _B)
    g, stats = _emit_call(emit_logits.reshape(_NT * _V), xt)
    root2 = root_logits.reshape(1, _NT)
    rule2 = rule_logits.transpose(0, 2, 1).reshape(_NT, _NT * _NT)
    ll = pl.pallas_call(
        _inside_kernel,
        out_shape=jax.ShapeDtypeStruct((_B, 1), jnp.float32),
        scratch_shapes=[
            pltpu.VMEM((_T + 1, (_T + 1) * _B, _NT * _NT), jnp.float32),
        ],
    )(g, stats, root2, rule2)
    return ll.reshape(_B)
